# initial kernel scaffold (unmeasured)
import jax
import jax.numpy as jnp
from jax import lax
from jax.experimental import pallas as pl
from jax.experimental.pallas import tpu as pltpu

N_DEV = 4

_sem_signal = getattr(pl, "semaphore_signal", None) or pltpu.semaphore_signal
_sem_wait = getattr(pl, "semaphore_wait", None) or pltpu.semaphore_wait
_DeviceIdType = getattr(pl, "DeviceIdType", None) or pltpu.DeviceIdType
_CompilerParams = getattr(pltpu, "CompilerParams", None) or pltpu.TPUCompilerParams


def kernel(x, Wq, K_ext, V_ext, Wo):
    B_loc, Sq, D = x.shape
    _, Hd = Wq.shape
    B, Skv, Hq, Dh = K_ext.shape
    HL = Hd // Dh
    BLK = 64

    def body(x_ref, wq_ref, k_hbm, v_hbm, wo_ref, out_ref,
             wq_comm, wo_comm, k_ch, v_ch,
             wq_ssem, wq_rsem, wo_ssem, wo_rsem, k_sem, v_sem):
        p = lax.axis_index("i")
        left = lax.rem(p - 1 + N_DEV, N_DEV)
        right = lax.rem(p + 1, N_DEV)

        kdmas, vdmas = [], []
        for h in range(N_DEV):
            q = lax.rem(p - h + N_DEV, N_DEV)
            for hh in range(HL):
                head = q * HL + hh
                s = h * HL + hh
                kd = pltpu.make_async_copy(
                    k_hbm.at[pl.ds(B_loc * p, B_loc), :, head, :],
                    k_ch.at[h, hh], k_sem.at[s])
                kd.start()
                kdmas.append(kd)
                vd = pltpu.make_async_copy(
                    v_hbm.at[pl.ds(B_loc * p, B_loc), :, head, :],
                    v_ch.at[h, hh], v_sem.at[s])
                vd.start()
                vdmas.append(vd)

        wq_comm[0] = wq_ref[...].astype(jnp.bfloat16)
        wo_comm[0] = wo_ref[...].astype(jnp.bfloat16)

        barrier_sem = pltpu.get_barrier_semaphore()
        for nbr in (left, right):
            _sem_signal(barrier_sem, inc=1, device_id=(nbr,),
                        device_id_type=_DeviceIdType.MESH)
        _sem_wait(barrier_sem, 2)

        x16 = x_ref[...].astype(jnp.bfloat16)
        ri = lax.broadcasted_iota(jnp.int32, (Sq, Skv), 0) // BLK
        ci = lax.broadcasted_iota(jnp.int32, (Sq, Skv), 1) // BLK
        mask = ci <= ri

        for h in range(N_DEV):
            if h < N_DEV - 1:
                wq_rdma = pltpu.make_async_remote_copy(
                    src_ref=wq_comm.at[h], dst_ref=wq_comm.at[h + 1],
                    send_sem=wq_ssem.at[h], recv_sem=wq_rsem.at[h],
                    device_id=(right,), device_id_type=_DeviceIdType.MESH)
                wq_rdma.start()
                wo_rdma = pltpu.make_async_remote_copy(
                    src_ref=wo_comm.at[h], dst_ref=wo_comm.at[h + 1],
                    send_sem=wo_ssem.at[h], recv_sem=wo_rsem.at[h],
                    device_id=(right,), device_id_type=_DeviceIdType.MESH)
                wo_rdma.start()

            for hh in range(HL):
                kdmas[h * HL + hh].wait()
                vdmas[h * HL + hh].wait()

            wq16 = wq_comm[h]
            wo16 = wo_comm[h]
            for b in range(B_loc):
                xb = x16[b]
                Qb = lax.dot_general(
                    xb, wq16, (((1,), (0,)), ((), ())),
                    preferred_element_type=jnp.float32).astype(jnp.bfloat16)
                ctxs = []
                for hh in range(HL):
                    qh = Qb[:, hh * Dh:(hh + 1) * Dh]
                    kh = k_ch[h, hh, b].astype(jnp.bfloat16)
                    vh = v_ch[h, hh, b].astype(jnp.bfloat16)
                    s = lax.dot_general(
                        qh, kh, (((1,), (1,)), ((), ())),
                        preferred_element_type=jnp.float32) * 0.125
                    s = jnp.where(mask, s, -1e9)
                    m = jnp.max(s, axis=-1, keepdims=True)
                    e = jnp.exp(s - m)
                    den = jnp.sum(e, axis=-1, keepdims=True)
                    w16 = (e / den).astype(jnp.bfloat16)
                    ctxs.append(lax.dot_general(
                        w16, vh, (((1,), (0,)), ((), ())),
                        preferred_element_type=jnp.float32))
                ctx = jnp.concatenate(ctxs, axis=1).astype(jnp.bfloat16)
                part = lax.dot_general(
                    ctx, wo16, (((1,), (0,)), ((), ())),
                    preferred_element_type=jnp.float32)
                if h == 0:
                    out_ref[b] = part
                else:
                    out_ref[b] = out_ref[b] + part

            if h < N_DEV - 1:
                wq_rdma.wait()
                wo_rdma.wait()

    vmem = pl.BlockSpec(memory_space=pltpu.VMEM)
    anym = pl.BlockSpec(memory_space=pltpu.ANY)
    return pl.pallas_call(
        body,
        out_shape=jax.ShapeDtypeStruct((B_loc, Sq, D), jnp.float32),
        in_specs=[vmem, vmem, anym, anym, vmem],
        out_specs=vmem,
        scratch_shapes=[
            pltpu.VMEM((N_DEV, D, Hd), jnp.bfloat16),
            pltpu.VMEM((N_DEV, Hd, D), jnp.bfloat16),
            pltpu.VMEM((N_DEV, HL, B_loc, Skv, Dh), jnp.float32),
            pltpu.VMEM((N_DEV, HL, B_loc, Skv, Dh), jnp.float32),
            pltpu.SemaphoreType.DMA((N_DEV - 1,)),
            pltpu.SemaphoreType.DMA((N_DEV - 1,)),
            pltpu.SemaphoreType.DMA((N_DEV - 1,)),
            pltpu.SemaphoreType.DMA((N_DEV - 1,)),
            pltpu.SemaphoreType.DMA((N_DEV * HL,)),
            pltpu.SemaphoreType.DMA((N_DEV * HL,)),
        ],
        compiler_params=_CompilerParams(collective_id=0),
    )(x, Wq, K_ext, V_ext, Wo)


# baseline (device time: 195536 ns/iter reference)
import jax
import jax.numpy as jnp
from jax import lax
from jax.experimental import pallas as pl
from jax.experimental.pallas import tpu as pltpu

N_DEV = 4

_sem_signal = getattr(pl, "semaphore_signal", None) or pltpu.semaphore_signal
_sem_wait = getattr(pl, "semaphore_wait", None) or pltpu.semaphore_wait
_DeviceIdType = getattr(pl, "DeviceIdType", None) or pltpu.DeviceIdType
_CompilerParams = getattr(pltpu, "CompilerParams", None) or pltpu.TPUCompilerParams


def kernel(x, Wq, K_ext, V_ext, Wo):
    B_loc, Sq, D = x.shape
    _, Hd = Wq.shape
    B, Skv, Hq, Dh = K_ext.shape
    HL = Hd // Dh
    BLK = 64

    def body(x_ref, wq_ref, k_hbm, v_hbm, wo_ref, out_ref,
             wq_comm, wo_comm, k_ch, v_ch,
             wq_ssem, wq_rsem, wo_ssem, wo_rsem, k_sem, v_sem):
        p = lax.axis_index("i")
        left = lax.rem(p - 1 + N_DEV, N_DEV)
        right = lax.rem(p + 1, N_DEV)

        kdmas, vdmas = [], []
        for h in range(N_DEV):
            q = lax.rem(p - h + N_DEV, N_DEV)
            for hh in range(HL):
                head = q * HL + hh
                s = h * HL + hh
                kd = pltpu.make_async_copy(
                    k_hbm.at[pl.ds(B_loc * p, B_loc), :, head, :],
                    k_ch.at[h, hh], k_sem.at[s])
                kd.start()
                kdmas.append(kd)
                vd = pltpu.make_async_copy(
                    v_hbm.at[pl.ds(B_loc * p, B_loc), :, head, :],
                    v_ch.at[h, hh], v_sem.at[s])
                vd.start()
                vdmas.append(vd)

        wq_comm[0] = wq_ref[...].astype(jnp.bfloat16)
        wo_comm[0] = wo_ref[...].astype(jnp.bfloat16)

        barrier_sem = pltpu.get_barrier_semaphore()
        for nbr in (left, right):
            _sem_signal(barrier_sem, inc=1, device_id=(nbr,),
                        device_id_type=_DeviceIdType.MESH)
        _sem_wait(barrier_sem, 2)

        x16 = x_ref[...].astype(jnp.bfloat16)
        ri = lax.broadcasted_iota(jnp.int32, (Sq, Skv), 0) // BLK
        ci = lax.broadcasted_iota(jnp.int32, (Sq, Skv), 1) // BLK
        mask = ci <= ri

        for h in range(N_DEV):
            if h < N_DEV - 1:
                wq_rdma = pltpu.make_async_remote_copy(
                    src_ref=wq_comm.at[h], dst_ref=wq_comm.at[h + 1],
                    send_sem=wq_ssem.at[h], recv_sem=wq_rsem.at[h],
                    device_id=(right,), device_id_type=_DeviceIdType.MESH)
                wq_rdma.start()
                wo_rdma = pltpu.make_async_remote_copy(
                    src_ref=wo_comm.at[h], dst_ref=wo_comm.at[h + 1],
                    send_sem=wo_ssem.at[h], recv_sem=wo_rsem.at[h],
                    device_id=(right,), device_id_type=_DeviceIdType.MESH)
                wo_rdma.start()

            for hh in range(HL):
                kdmas[h * HL + hh].wait()
                vdmas[h * HL + hh].wait()

            wq16 = wq_comm[h]
            wo16 = wo_comm[h]
            for b in range(B_loc):
                xb = x16[b]
                Qb = lax.dot_general(
                    xb, wq16, (((1,), (0,)), ((), ())),
                    preferred_element_type=jnp.float32).astype(jnp.bfloat16)
                ctxs = []
                for hh in range(HL):
                    qh = Qb[:, hh * Dh:(hh + 1) * Dh]
                    kh = k_ch[h, hh, b].astype(jnp.bfloat16)
                    vh = v_ch[h, hh, b].astype(jnp.bfloat16)
                    s = lax.dot_general(
                        qh, kh, (((1,), (1,)), ((), ())),
                        preferred_element_type=jnp.float32) * 0.125
                    s = jnp.where(mask, s, -1e9)
                    m = jnp.max(s, axis=-1, keepdims=True)
                    e = jnp.exp(s - m)
                    den = jnp.sum(e, axis=-1, keepdims=True)
                    w16 = (e / den).astype(jnp.bfloat16)
                    ctxs.append(lax.dot_general(
                        w16, vh, (((1,), (0,)), ((), ())),
                        preferred_element_type=jnp.float32))
                ctx = jnp.concatenate(ctxs, axis=1).astype(jnp.bfloat16)
                part = lax.dot_general(
                    ctx, wo16, (((1,), (0,)), ((), ())),
                    preferred_element_type=jnp.float32)
                if h == 0:
                    out_ref[b] = part
                else:
                    out_ref[b] = out_ref[b] + part

            if h < N_DEV - 1:
                wq_rdma.wait()
                wo_rdma.wait()

    vmem = pl.BlockSpec(memory_space=pltpu.VMEM)
    anym = pl.BlockSpec(memory_space=pl.ANY)
    return pl.pallas_call(
        body,
        out_shape=jax.ShapeDtypeStruct((B_loc, Sq, D), jnp.float32),
        in_specs=[vmem, vmem, anym, anym, vmem],
        out_specs=vmem,
        scratch_shapes=[
            pltpu.VMEM((N_DEV, D, Hd), jnp.bfloat16),
            pltpu.VMEM((N_DEV, Hd, D), jnp.bfloat16),
            pltpu.VMEM((N_DEV, HL, B_loc, Skv, Dh), jnp.float32),
            pltpu.VMEM((N_DEV, HL, B_loc, Skv, Dh), jnp.float32),
            pltpu.SemaphoreType.DMA((N_DEV - 1,)),
            pltpu.SemaphoreType.DMA((N_DEV - 1,)),
            pltpu.SemaphoreType.DMA((N_DEV - 1,)),
            pltpu.SemaphoreType.DMA((N_DEV - 1,)),
            pltpu.SemaphoreType.DMA((N_DEV * HL,)),
            pltpu.SemaphoreType.DMA((N_DEV * HL,)),
        ],
        compiler_params=_CompilerParams(
            collective_id=0, vmem_limit_bytes=100 * 1024 * 1024),
    )(x, Wq, K_ext, V_ext, Wo)


# device time: 194100 ns/iter; 1.0074x vs baseline; 1.0074x over previous
import jax
import jax.numpy as jnp
from jax import lax
from jax.experimental import pallas as pl
from jax.experimental.pallas import tpu as pltpu

N_DEV = 4

_sem_signal = getattr(pl, "semaphore_signal", None) or pltpu.semaphore_signal
_sem_wait = getattr(pl, "semaphore_wait", None) or pltpu.semaphore_wait
_DeviceIdType = getattr(pl, "DeviceIdType", None) or pltpu.DeviceIdType
_CompilerParams = getattr(pltpu, "CompilerParams", None) or pltpu.TPUCompilerParams


def kernel(x, Wq, K_ext, V_ext, Wo):
    B_loc, Sq, D = x.shape
    _, Hd = Wq.shape
    B, Skv, Hq, Dh = K_ext.shape
    HL = Hd // Dh
    BLK = 64

    def body(x_ref, wq_ref, k_hbm, v_hbm, wo_ref, out_ref,
             wq_comm, wo_comm, k_ch, v_ch,
             wq_ssem, wq_rsem, wo_ssem, wo_rsem, k_sem, v_sem):
        p = lax.axis_index("i")
        left = lax.rem(p - 1 + N_DEV, N_DEV)
        right = lax.rem(p + 1, N_DEV)

        kdmas, vdmas = [], []
        for h in range(N_DEV):
            q = lax.rem(p - h + N_DEV, N_DEV)
            for hh in range(HL):
                head = q * HL + hh
                s = h * HL + hh
                kd = pltpu.make_async_copy(
                    k_hbm.at[pl.ds(B_loc * p, B_loc), :, head, :],
                    k_ch.at[h, hh], k_sem.at[s])
                kd.start()
                kdmas.append(kd)
                vd = pltpu.make_async_copy(
                    v_hbm.at[pl.ds(B_loc * p, B_loc), :, head, :],
                    v_ch.at[h, hh], v_sem.at[s])
                vd.start()
                vdmas.append(vd)

        wq_comm[0] = wq_ref[...].astype(jnp.bfloat16)
        wo_comm[0] = wo_ref[...].astype(jnp.bfloat16)

        barrier_sem = pltpu.get_barrier_semaphore()
        for nbr in (left, right):
            _sem_signal(barrier_sem, inc=1, device_id=(nbr,),
                        device_id_type=_DeviceIdType.MESH)
        _sem_wait(barrier_sem, 2)

        x16 = x_ref[...].astype(jnp.bfloat16)
        ri = lax.broadcasted_iota(jnp.int32, (Sq, Skv), 0) // BLK
        ci = lax.broadcasted_iota(jnp.int32, (Sq, Skv), 1) // BLK
        mask16 = (ci <= ri).astype(jnp.bfloat16)

        for h in range(N_DEV):
            if h < N_DEV - 1:
                wq_rdma = pltpu.make_async_remote_copy(
                    src_ref=wq_comm.at[h], dst_ref=wq_comm.at[h + 1],
                    send_sem=wq_ssem.at[h], recv_sem=wq_rsem.at[h],
                    device_id=(right,), device_id_type=_DeviceIdType.MESH)
                wq_rdma.start()
                wo_rdma = pltpu.make_async_remote_copy(
                    src_ref=wo_comm.at[h], dst_ref=wo_comm.at[h + 1],
                    send_sem=wo_ssem.at[h], recv_sem=wo_rsem.at[h],
                    device_id=(right,), device_id_type=_DeviceIdType.MESH)
                wo_rdma.start()

            for hh in range(HL):
                kdmas[h * HL + hh].wait()
                vdmas[h * HL + hh].wait()

            wq16 = wq_comm[h]
            wo16 = wo_comm[h]
            for b in range(B_loc):
                xb = x16[b]
                Qb = (lax.dot_general(
                    xb, wq16, (((1,), (0,)), ((), ())),
                    preferred_element_type=jnp.float32)
                    * 0.125).astype(jnp.bfloat16)
                ctxs = []
                for hh in range(HL):
                    qh = Qb[:, hh * Dh:(hh + 1) * Dh]
                    kh = k_ch[h, hh, b].astype(jnp.bfloat16)
                    vh = v_ch[h, hh, b].astype(jnp.bfloat16)
                    s = lax.dot_general(
                        qh, kh, (((1,), (1,)), ((), ())),
                        preferred_element_type=jnp.float32)
                    e = jnp.exp(s).astype(jnp.bfloat16) * mask16
                    den = jnp.sum(e, axis=-1, keepdims=True,
                                  dtype=jnp.float32)
                    ctx_h = lax.dot_general(
                        e, vh, (((1,), (0,)), ((), ())),
                        preferred_element_type=jnp.float32)
                    ctxs.append(ctx_h / den)
                ctx = jnp.concatenate(ctxs, axis=1).astype(jnp.bfloat16)
                part = lax.dot_general(
                    ctx, wo16, (((1,), (0,)), ((), ())),
                    preferred_element_type=jnp.float32)
                if h == 0:
                    out_ref[b] = part
                else:
                    out_ref[b] = out_ref[b] + part

            if h < N_DEV - 1:
                wq_rdma.wait()
                wo_rdma.wait()

    vmem = pl.BlockSpec(memory_space=pltpu.VMEM)
    anym = pl.BlockSpec(memory_space=pl.ANY)
    return pl.pallas_call(
        body,
        out_shape=jax.ShapeDtypeStruct((B_loc, Sq, D), jnp.float32),
        in_specs=[vmem, vmem, anym, anym, vmem],
        out_specs=vmem,
        scratch_shapes=[
            pltpu.VMEM((N_DEV, D, Hd), jnp.bfloat16),
            pltpu.VMEM((N_DEV, Hd, D), jnp.bfloat16),
            pltpu.VMEM((N_DEV, HL, B_loc, Skv, Dh), jnp.float32),
            pltpu.VMEM((N_DEV, HL, B_loc, Skv, Dh), jnp.float32),
            pltpu.SemaphoreType.DMA((N_DEV - 1,)),
            pltpu.SemaphoreType.DMA((N_DEV - 1,)),
            pltpu.SemaphoreType.DMA((N_DEV - 1,)),
            pltpu.SemaphoreType.DMA((N_DEV - 1,)),
            pltpu.SemaphoreType.DMA((N_DEV * HL,)),
            pltpu.SemaphoreType.DMA((N_DEV * HL,)),
        ],
        compiler_params=_CompilerParams(
            collective_id=0, vmem_limit_bytes=100 * 1024 * 1024),
    )(x, Wq, K_ext, V_ext, Wo)
